# baseline (speedup 1.0000x reference)
import functools

import numpy as np

import jax
import jax.numpy as jnp
from jax import lax
from jax.experimental import pallas as pl
from jax.experimental.pallas import tpu as pltpu
from jax.experimental.pallas import tpu_sc as plsc

B = 1024
N = 10000
E = 160000

NC = 2
NS = 16
NW = NC * NS
P = B // 2
PROWS_PER_W = P // NW
SLAB = 8
N_SLABS = PROWS_PER_W // SLAB
ROWS_PER_W = B // NW
CHUNK = 4000
N_PAIRS = E // (2 * CHUNK)
GROUPS = CHUNK // 16
H_GROUPS = N // 16

HIMASK = -65536

_mesh = plsc.VectorSubcoreMesh(core_axis_name="c", subcore_axis_name="s")


@functools.partial(
    pl.kernel,
    mesh=_mesh,
    compiler_params=pltpu.CompilerParams(needs_layout_passes=False),
    out_type=jax.ShapeDtypeStruct((B,), jnp.float32),
    scratch_types=[
        pltpu.VMEM((SLAB * N,), jnp.int32),
        pltpu.VMEM((N,), jnp.float32),
        pltpu.VMEM((CHUNK,), jnp.int32),
        pltpu.VMEM((CHUNK,), jnp.int32),
        pltpu.VMEM((CHUNK,), jnp.float32),
        pltpu.VMEM((CHUNK,), jnp.int32),
        pltpu.VMEM((CHUNK,), jnp.int32),
        pltpu.VMEM((CHUNK,), jnp.float32),
        pltpu.VMEM((ROWS_PER_W,), jnp.float32),
        pltpu.SemaphoreType.DMA,
        pltpu.SemaphoreType.DMA,
    ],
)
def _rbm_sc(xp_hbm, h_hbm, j_hbm, ei_hbm, ej_hbm, out_hbm,
            xslab, h_v, ei_a, ej_a, jv_a, ei_b, ej_b, jv_b, out_v,
            sem_a, sem_b):
    wid = lax.axis_index("s") * NC + lax.axis_index("c")

    pltpu.sync_copy(h_hbm, h_v)

    lane = lax.iota(jnp.int32, 16)
    row_refs = [xslab.at[pl.ds(r * N, N)] for r in range(SLAB)]

    def start_chunk(c, bufs, sem):
        off = c * CHUNK
        pltpu.async_copy(ei_hbm.at[pl.ds(off, CHUNK)], bufs[0], sem)
        pltpu.async_copy(ej_hbm.at[pl.ds(off, CHUNK)], bufs[1], sem)
        pltpu.async_copy(j_hbm.at[pl.ds(off, CHUNK)], bufs[2], sem)

    def wait_chunk(bufs, sem):
        pltpu.make_async_copy(ei_hbm.at[pl.ds(0, CHUNK)], bufs[0], sem).wait()
        pltpu.make_async_copy(ej_hbm.at[pl.ds(0, CHUNK)], bufs[1], sem).wait()
        pltpu.make_async_copy(j_hbm.at[pl.ds(0, CHUNK)], bufs[2], sem).wait()

    def edge_accum(bufs, accs):
        ei_v, ej_v, jv_v = bufs

        def group_body(g, accs):
            base = g * 16
            ii = ei_v[pl.ds(base, 16)]
            jj = ej_v[pl.ds(base, 16)]
            Jv = jv_v[pl.ds(base, 16)]
            acc_a, acc_b = accs
            new_a, new_b = [], []
            for r in range(SLAB):
                wi = plsc.load_gather(row_refs[r], [ii])
                wj = plsc.load_gather(row_refs[r], [jj])
                ai = plsc.bitcast(wi & HIMASK, jnp.float32)
                aj = plsc.bitcast(wj & HIMASK, jnp.float32)
                bi = plsc.bitcast(wi << 16, jnp.float32)
                bj = plsc.bitcast(wj << 16, jnp.float32)
                new_a.append(acc_a[r] + (ai * aj) * Jv)
                new_b.append(acc_b[r] + (bi * bj) * Jv)
            return tuple(new_a), tuple(new_b)

        return plsc.parallel_loop(0, GROUPS, unroll=2, carry=accs)(group_body)

    bufs_a = (ei_a, ej_a, jv_a)
    bufs_b = (ei_b, ej_b, jv_b)

    tile_sums = [None] * ROWS_PER_W
    for s in range(N_SLABS):
        prow0 = wid * PROWS_PER_W + s * SLAB
        pltpu.async_copy(xp_hbm.at[pl.ds(prow0 * N, SLAB * N)], xslab, sem_a)
        start_chunk(0, bufs_a, sem_a)
        pltpu.make_async_copy(
            xp_hbm.at[pl.ds(0, SLAB * N)], xslab, sem_a).wait()

        def h_body(k, accs):
            base = k * 16
            hv = h_v[pl.ds(base, 16)]
            acc_a, acc_b = accs
            new_a, new_b = [], []
            for r in range(SLAB):
                w = xslab[pl.ds(r * N + base, 16)]
                av = plsc.bitcast(w & HIMASK, jnp.float32)
                bv = plsc.bitcast(w << 16, jnp.float32)
                new_a.append(acc_a[r] + av * hv)
                new_b.append(acc_b[r] + bv * hv)
            return tuple(new_a), tuple(new_b)

        zeros = tuple(jnp.zeros((16,), jnp.float32) for _ in range(SLAB))
        accs = lax.fori_loop(0, H_GROUPS, h_body, (zeros, zeros))

        def pair_body(c, accs):
            start_chunk(2 * c + 1, bufs_b, sem_b)
            wait_chunk(bufs_a, sem_a)
            accs = edge_accum(bufs_a, accs)

            @pl.when(c < N_PAIRS - 1)
            def _():
                start_chunk(2 * c + 2, bufs_a, sem_a)

            wait_chunk(bufs_b, sem_b)
            return edge_accum(bufs_b, accs)

        acc_a, acc_b = lax.fori_loop(0, N_PAIRS, pair_body, accs)
        for r in range(SLAB):
            tile_sums[16 * s + 2 * r] = lax.reduce_sum_p.bind(
                acc_a[r], axes=(0,))
            tile_sums[16 * s + 2 * r + 1] = lax.reduce_sum_p.bind(
                acc_b[r], axes=(0,))

    for half in range(ROWS_PER_W // 16):
        vec = jnp.zeros((16,), jnp.float32)
        for k in range(16):
            vec = jnp.where(lane == k, tile_sums[half * 16 + k], vec)
        out_v[pl.ds(half * 16, 16)] = vec
    pltpu.sync_copy(out_v, out_hbm.at[pl.ds(wid * ROWS_PER_W, ROWS_PER_W)])


def kernel(x, h, J, edge_idx_i, edge_idx_j):
    bits = lax.bitcast_convert_type(x, jnp.uint32)
    rounded = (bits + np.uint32(0x8000)) & np.uint32(0xFFFF0000)
    hi = rounded[0::2]
    lo = rounded[1::2] >> np.uint32(16)
    xp = lax.bitcast_convert_type(hi | lo, jnp.int32).reshape(-1)
    return _rbm_sc(xp, h, J, edge_idx_i, edge_idx_j)


# speedup vs baseline: 2.1922x; 2.1922x over previous
import functools

import numpy as np

import jax
import jax.numpy as jnp
from jax import lax
from jax.experimental import pallas as pl
from jax.experimental.pallas import tpu as pltpu
from jax.experimental.pallas import tpu_sc as plsc

B = 1024
N = 10000
E = 160000

NC = 2
NS = 16
NW = NC * NS
P = B // 2
PROWS_PER_W = P // NW
SLAB = 8
N_SLABS = PROWS_PER_W // SLAB
ROWS_PER_W = B // NW
CHUNK = 4000
N_PAIRS = E // (2 * CHUNK)
GROUPS = CHUNK // 16
H_GROUPS = N // 16

HIMASK = -65536

_mesh = plsc.VectorSubcoreMesh(core_axis_name="c", subcore_axis_name="s")


@functools.partial(
    pl.kernel,
    mesh=_mesh,
    compiler_params=pltpu.CompilerParams(needs_layout_passes=False),
    out_type=jax.ShapeDtypeStruct((B,), jnp.float32),
    scratch_types=[
        pltpu.VMEM((SLAB * N,), jnp.int32),
        pltpu.VMEM((N,), jnp.float32),
        pltpu.VMEM((CHUNK,), jnp.int32),
        pltpu.VMEM((CHUNK,), jnp.int32),
        pltpu.VMEM((CHUNK,), jnp.float32),
        pltpu.VMEM((CHUNK,), jnp.int32),
        pltpu.VMEM((CHUNK,), jnp.int32),
        pltpu.VMEM((CHUNK,), jnp.float32),
        pltpu.VMEM((ROWS_PER_W,), jnp.float32),
        pltpu.SemaphoreType.DMA,
        pltpu.SemaphoreType.DMA,
    ],
)
def _rbm_sc(xp_hbm, h_hbm, j_hbm, ei_hbm, ej_hbm, out_hbm,
            xslab, h_v, ei_a, ej_a, jv_a, ei_b, ej_b, jv_b, out_v,
            sem_a, sem_b):
    wid = lax.axis_index("s") * NC + lax.axis_index("c")

    pltpu.sync_copy(h_hbm, h_v)

    lane = lax.iota(jnp.int32, 16)
    row_refs = [xslab.at[pl.ds(r * N, N)] for r in range(SLAB)]

    def start_chunk(c, bufs, sem):
        off = c * CHUNK
        pltpu.async_copy(ei_hbm.at[pl.ds(off, CHUNK)], bufs[0], sem)
        pltpu.async_copy(ej_hbm.at[pl.ds(off, CHUNK)], bufs[1], sem)
        pltpu.async_copy(j_hbm.at[pl.ds(off, CHUNK)], bufs[2], sem)

    def wait_chunk(bufs, sem):
        pltpu.make_async_copy(ei_hbm.at[pl.ds(0, CHUNK)], bufs[0], sem).wait()
        pltpu.make_async_copy(ej_hbm.at[pl.ds(0, CHUNK)], bufs[1], sem).wait()
        pltpu.make_async_copy(j_hbm.at[pl.ds(0, CHUNK)], bufs[2], sem).wait()

    def edge_accum(bufs, accs):
        ei_v, ej_v, jv_v = bufs

        def group_body(g, accs):
            base = g * 16
            ii = ei_v[pl.ds(base, 16)]
            jj = ej_v[pl.ds(base, 16)]
            Jv = jv_v[pl.ds(base, 16)]
            acc_a, acc_b = accs
            new_a, new_b = [], []
            for r in range(SLAB):
                wi = plsc.load_gather(row_refs[r], [ii])
                wj = plsc.load_gather(row_refs[r], [jj])
                ai = plsc.bitcast(wi & HIMASK, jnp.float32)
                aj = plsc.bitcast(wj & HIMASK, jnp.float32)
                bi = plsc.bitcast(wi << 16, jnp.float32)
                bj = plsc.bitcast(wj << 16, jnp.float32)
                new_a.append(acc_a[r] + (ai * aj) * Jv)
                new_b.append(acc_b[r] + (bi * bj) * Jv)
            return tuple(new_a), tuple(new_b)

        return plsc.parallel_loop(0, GROUPS, unroll=2, carry=accs)(group_body)

    bufs_a = (ei_a, ej_a, jv_a)
    bufs_b = (ei_b, ej_b, jv_b)

    tile_sums = [None] * ROWS_PER_W
    for s in range(N_SLABS):
        prow0 = wid * PROWS_PER_W + s * SLAB
        pltpu.async_copy(xp_hbm.at[pl.ds(prow0 * N, SLAB * N)], xslab, sem_a)
        start_chunk(0, bufs_a, sem_a)
        pltpu.make_async_copy(
            xp_hbm.at[pl.ds(0, SLAB * N)], xslab, sem_a).wait()

        def h_body(k, accs):
            base = k * 16
            hv = h_v[pl.ds(base, 16)]
            acc_a, acc_b = accs
            new_a, new_b = [], []
            for r in range(SLAB):
                w = xslab[pl.ds(r * N + base, 16)]
                av = plsc.bitcast(w & HIMASK, jnp.float32)
                bv = plsc.bitcast(w << 16, jnp.float32)
                new_a.append(acc_a[r] + av * hv)
                new_b.append(acc_b[r] + bv * hv)
            return tuple(new_a), tuple(new_b)

        zeros = tuple(jnp.zeros((16,), jnp.float32) for _ in range(SLAB))
        accs = lax.fori_loop(0, H_GROUPS, h_body, (zeros, zeros))

        def pair_body(c, accs):
            start_chunk(2 * c + 1, bufs_b, sem_b)
            wait_chunk(bufs_a, sem_a)
            accs = edge_accum(bufs_a, accs)

            @pl.when(c < N_PAIRS - 1)
            def _():
                start_chunk(2 * c + 2, bufs_a, sem_a)

            wait_chunk(bufs_b, sem_b)
            return edge_accum(bufs_b, accs)

        acc_a, acc_b = lax.fori_loop(0, N_PAIRS, pair_body, accs)
        for r in range(SLAB):
            tile_sums[16 * s + 2 * r] = lax.reduce_sum_p.bind(
                acc_a[r], axes=(0,))
            tile_sums[16 * s + 2 * r + 1] = lax.reduce_sum_p.bind(
                acc_b[r], axes=(0,))

    for half in range(ROWS_PER_W // 16):
        vec = jnp.zeros((16,), jnp.float32)
        for k in range(16):
            vec = jnp.where(lane == k, tile_sums[half * 16 + k], vec)
        out_v[pl.ds(half * 16, 16)] = vec
    pltpu.sync_copy(out_v, out_hbm.at[pl.ds(wid * ROWS_PER_W, ROWS_PER_W)])


def kernel(x, h, J, edge_idx_i, edge_idx_j):
    bits = lax.bitcast_convert_type(x.reshape(P, 2, N), jnp.uint32)
    rounded = (bits + np.uint32(0x8000)) & np.uint32(0xFFFF0000)
    hi = rounded[:, 0, :]
    lo = rounded[:, 1, :] >> np.uint32(16)
    xp = lax.bitcast_convert_type(hi | lo, jnp.int32).reshape(-1)
    return _rbm_sc(xp, h, J, edge_idx_i, edge_idx_j)


# speedup vs baseline: 3.1019x; 1.4150x over previous
import functools

import jax
import jax.numpy as jnp
from jax import lax
from jax.experimental import pallas as pl
from jax.experimental.pallas import tpu as pltpu
from jax.experimental.pallas import tpu_sc as plsc

B = 1024
N = 10000
E = 160000

NC = 2
NS = 16
NW = NC * NS
P = B // 2
PROWS_PER_W = P // NW
SLAB = 8
N_SLABS = PROWS_PER_W // SLAB
ROWS_PER_W = B // NW
CHUNK = 3200
N_PAIRS = E // (2 * CHUNK)
GROUPS = CHUNK // 16
H_GROUPS = N // 16

HIMASK = -65536

_mesh = plsc.VectorSubcoreMesh(core_axis_name="c", subcore_axis_name="s")


@functools.partial(
    pl.kernel,
    mesh=_mesh,
    compiler_params=pltpu.CompilerParams(needs_layout_passes=False),
    out_type=jax.ShapeDtypeStruct((B,), jnp.float32),
    scratch_types=[
        pltpu.VMEM((SLAB * N,), jnp.int32),
        pltpu.VMEM((N,), jnp.float32),
        pltpu.VMEM((N,), jnp.float32),
        pltpu.VMEM((N,), jnp.float32),
        pltpu.VMEM((CHUNK,), jnp.int32),
        pltpu.VMEM((CHUNK,), jnp.int32),
        pltpu.VMEM((CHUNK,), jnp.float32),
        pltpu.VMEM((CHUNK,), jnp.int32),
        pltpu.VMEM((CHUNK,), jnp.int32),
        pltpu.VMEM((CHUNK,), jnp.float32),
        pltpu.VMEM((ROWS_PER_W,), jnp.float32),
        pltpu.SemaphoreType.DMA,
        pltpu.SemaphoreType.DMA,
        pltpu.SemaphoreType.DMA,
    ],
)
def _rbm_sc(x_hbm, h_hbm, j_hbm, ei_hbm, ej_hbm, out_hbm,
            xslab, stg0, stg1, h_v, ei_a, ej_a, jv_a, ei_b, ej_b, jv_b,
            out_v, sem_a, sem_b, sem_s):
    wid = lax.axis_index("s") * NC + lax.axis_index("c")

    pltpu.sync_copy(h_hbm, h_v)

    lane = lax.iota(jnp.int32, 16)
    row_refs = [xslab.at[pl.ds(r * N, N)] for r in range(SLAB)]
    stg = (stg0, stg1)

    def start_row(batch_row, buf):
        pltpu.async_copy(x_hbm.at[pl.ds(batch_row * N, N)], buf, sem_s)

    def wait_row(buf):
        pltpu.make_async_copy(x_hbm.at[pl.ds(0, N)], buf, sem_s).wait()

    def pack_hi(src, r):
        def body(k, _):
            o = k * 16
            w = plsc.bitcast(src[pl.ds(o, 16)], jnp.int32)
            xslab[pl.ds(r * N + o, 16)] = (w + 0x8000) & HIMASK
            return 0

        lax.fori_loop(0, H_GROUPS, body, 0)

    def pack_lo(src, r):
        def body(k, _):
            o = k * 16
            w = plsc.bitcast(src[pl.ds(o, 16)], jnp.int32)
            t = lax.shift_right_logical(w + 0x8000, 16)
            xslab[pl.ds(r * N + o, 16)] |= t
            return 0

        lax.fori_loop(0, H_GROUPS, body, 0)

    def start_chunk(c, bufs, sem):
        off = c * CHUNK
        pltpu.async_copy(ei_hbm.at[pl.ds(off, CHUNK)], bufs[0], sem)
        pltpu.async_copy(ej_hbm.at[pl.ds(off, CHUNK)], bufs[1], sem)
        pltpu.async_copy(j_hbm.at[pl.ds(off, CHUNK)], bufs[2], sem)

    def wait_chunk(bufs, sem):
        pltpu.make_async_copy(ei_hbm.at[pl.ds(0, CHUNK)], bufs[0], sem).wait()
        pltpu.make_async_copy(ej_hbm.at[pl.ds(0, CHUNK)], bufs[1], sem).wait()
        pltpu.make_async_copy(j_hbm.at[pl.ds(0, CHUNK)], bufs[2], sem).wait()

    def edge_accum(bufs, accs):
        ei_v, ej_v, jv_v = bufs

        def group_body(g, accs):
            base = g * 16
            ii = ei_v[pl.ds(base, 16)]
            jj = ej_v[pl.ds(base, 16)]
            Jv = jv_v[pl.ds(base, 16)]
            acc_a, acc_b = accs
            new_a, new_b = [], []
            for r in range(SLAB):
                wi = plsc.load_gather(row_refs[r], [ii])
                wj = plsc.load_gather(row_refs[r], [jj])
                ai = plsc.bitcast(wi & HIMASK, jnp.float32)
                aj = plsc.bitcast(wj & HIMASK, jnp.float32)
                bi = plsc.bitcast(wi << 16, jnp.float32)
                bj = plsc.bitcast(wj << 16, jnp.float32)
                new_a.append(acc_a[r] + (ai * aj) * Jv)
                new_b.append(acc_b[r] + (bi * bj) * Jv)
            return tuple(new_a), tuple(new_b)

        return plsc.parallel_loop(0, GROUPS, unroll=2, carry=accs)(group_body)

    bufs_a = (ei_a, ej_a, jv_a)
    bufs_b = (ei_b, ej_b, jv_b)

    tile_sums = [None] * ROWS_PER_W
    for s in range(N_SLABS):
        p0 = wid * PROWS_PER_W + s * SLAB
        start_chunk(0, bufs_a, sem_a)
        start_row(2 * p0, stg[0])
        for r in range(SLAB):
            p = p0 + r
            wait_row(stg[0])
            start_row(2 * p + 1, stg[1])
            pack_hi(stg[0], r)
            wait_row(stg[1])
            if r < SLAB - 1:
                start_row(2 * p + 2, stg[0])
            pack_lo(stg[1], r)

        def h_body(k, accs):
            base = k * 16
            hv = h_v[pl.ds(base, 16)]
            acc_a, acc_b = accs
            new_a, new_b = [], []
            for r in range(SLAB):
                w = xslab[pl.ds(r * N + base, 16)]
                av = plsc.bitcast(w & HIMASK, jnp.float32)
                bv = plsc.bitcast(w << 16, jnp.float32)
                new_a.append(acc_a[r] + av * hv)
                new_b.append(acc_b[r] + bv * hv)
            return tuple(new_a), tuple(new_b)

        zeros = tuple(jnp.zeros((16,), jnp.float32) for _ in range(SLAB))
        accs = lax.fori_loop(0, H_GROUPS, h_body, (zeros, zeros))

        def pair_body(c, accs):
            start_chunk(2 * c + 1, bufs_b, sem_b)
            wait_chunk(bufs_a, sem_a)
            accs = edge_accum(bufs_a, accs)

            @pl.when(c < N_PAIRS - 1)
            def _():
                start_chunk(2 * c + 2, bufs_a, sem_a)

            wait_chunk(bufs_b, sem_b)
            return edge_accum(bufs_b, accs)

        acc_a, acc_b = lax.fori_loop(0, N_PAIRS, pair_body, accs)
        for r in range(SLAB):
            tile_sums[16 * s + 2 * r] = lax.reduce_sum_p.bind(
                acc_a[r], axes=(0,))
            tile_sums[16 * s + 2 * r + 1] = lax.reduce_sum_p.bind(
                acc_b[r], axes=(0,))

    for half in range(ROWS_PER_W // 16):
        vec = jnp.zeros((16,), jnp.float32)
        for k in range(16):
            vec = jnp.where(lane == k, tile_sums[half * 16 + k], vec)
        out_v[pl.ds(half * 16, 16)] = vec
    pltpu.sync_copy(out_v, out_hbm.at[pl.ds(wid * ROWS_PER_W, ROWS_PER_W)])


def kernel(x, h, J, edge_idx_i, edge_idx_j):
    return _rbm_sc(x.reshape(-1), h, J, edge_idx_i, edge_idx_j)


# speedup vs baseline: 3.3490x; 1.0797x over previous
import functools

import jax
import jax.numpy as jnp
from jax import lax
from jax.experimental import pallas as pl
from jax.experimental.pallas import tpu as pltpu
from jax.experimental.pallas import tpu_sc as plsc

B = 1024
N = 10000
E = 160000

NC = 2
NS = 16
NW = NC * NS
P = B // 2
PROWS_PER_W = P // NW
SLAB = 8
N_SLABS = PROWS_PER_W // SLAB
ROWS_PER_W = B // NW
CHUNK = 3200
N_PAIRS = E // (2 * CHUNK)
GROUPS = CHUNK // 16
H_GROUPS = N // 16

HIMASK = -65536

_mesh = plsc.VectorSubcoreMesh(core_axis_name="c", subcore_axis_name="s")


@functools.partial(
    pl.kernel,
    mesh=_mesh,
    compiler_params=pltpu.CompilerParams(needs_layout_passes=False),
    out_type=jax.ShapeDtypeStruct((B,), jnp.float32),
    scratch_types=[
        pltpu.VMEM((SLAB * N,), jnp.int32),
        pltpu.VMEM((N,), jnp.float32),
        pltpu.VMEM((N,), jnp.float32),
        pltpu.VMEM((N,), jnp.float32),
        pltpu.VMEM((CHUNK,), jnp.int32),
        pltpu.VMEM((CHUNK,), jnp.int32),
        pltpu.VMEM((CHUNK,), jnp.float32),
        pltpu.VMEM((CHUNK,), jnp.int32),
        pltpu.VMEM((CHUNK,), jnp.int32),
        pltpu.VMEM((CHUNK,), jnp.float32),
        pltpu.VMEM((ROWS_PER_W,), jnp.float32),
        pltpu.SemaphoreType.DMA,
        pltpu.SemaphoreType.DMA,
        pltpu.SemaphoreType.DMA,
    ],
)
def _rbm_sc(x_hbm, h_hbm, j_hbm, ei_hbm, ej_hbm, out_hbm,
            xslab, stg0, stg1, h_v, ei_a, ej_a, jv_a, ei_b, ej_b, jv_b,
            out_v, sem_a, sem_b, sem_s):
    wid = lax.axis_index("s") * NC + lax.axis_index("c")

    pltpu.sync_copy(h_hbm, h_v)

    lane = lax.iota(jnp.int32, 16)
    row_refs = [xslab.at[pl.ds(r * N, N)] for r in range(SLAB)]
    stg = (stg0, stg1)

    def start_row(batch_row, buf):
        pltpu.async_copy(x_hbm.at[pl.ds(batch_row * N, N)], buf, sem_s)

    def wait_row(buf):
        pltpu.make_async_copy(x_hbm.at[pl.ds(0, N)], buf, sem_s).wait()

    def pack_hi(src, r):
        def body(k):
            o = k * 16
            w = plsc.bitcast(src[pl.ds(o, 16)], jnp.int32)
            xslab[pl.ds(r * N + o, 16)] = (w + 0x8000) & HIMASK

        plsc.parallel_loop(0, H_GROUPS, unroll=4)(body)

    def pack_lo(src, r):
        def body(k):
            o = k * 16
            w = plsc.bitcast(src[pl.ds(o, 16)], jnp.int32)
            t = lax.shift_right_logical(w + 0x8000, 16)
            xslab[pl.ds(r * N + o, 16)] |= t

        plsc.parallel_loop(0, H_GROUPS, unroll=4)(body)

    def start_chunk(c, bufs, sem):
        off = c * CHUNK
        pltpu.async_copy(ei_hbm.at[pl.ds(off, CHUNK)], bufs[0], sem)
        pltpu.async_copy(ej_hbm.at[pl.ds(off, CHUNK)], bufs[1], sem)
        pltpu.async_copy(j_hbm.at[pl.ds(off, CHUNK)], bufs[2], sem)

    def wait_chunk(bufs, sem):
        pltpu.make_async_copy(ei_hbm.at[pl.ds(0, CHUNK)], bufs[0], sem).wait()
        pltpu.make_async_copy(ej_hbm.at[pl.ds(0, CHUNK)], bufs[1], sem).wait()
        pltpu.make_async_copy(j_hbm.at[pl.ds(0, CHUNK)], bufs[2], sem).wait()

    def edge_accum(bufs, accs):
        ei_v, ej_v, jv_v = bufs

        def group_body(g, accs):
            base = g * 16
            ii = ei_v[pl.ds(base, 16)]
            jj = ej_v[pl.ds(base, 16)]
            Jv = jv_v[pl.ds(base, 16)]
            acc_a, acc_b = accs
            new_a, new_b = [], []
            for r in range(SLAB):
                wi = plsc.load_gather(row_refs[r], [ii])
                wj = plsc.load_gather(row_refs[r], [jj])
                ai = plsc.bitcast(wi & HIMASK, jnp.float32)
                aj = plsc.bitcast(wj & HIMASK, jnp.float32)
                bi = plsc.bitcast(wi << 16, jnp.float32)
                bj = plsc.bitcast(wj << 16, jnp.float32)
                new_a.append(acc_a[r] + (ai * aj) * Jv)
                new_b.append(acc_b[r] + (bi * bj) * Jv)
            return tuple(new_a), tuple(new_b)

        return plsc.parallel_loop(0, GROUPS, unroll=2, carry=accs)(group_body)

    bufs_a = (ei_a, ej_a, jv_a)
    bufs_b = (ei_b, ej_b, jv_b)

    tile_sums = [None] * ROWS_PER_W
    for s in range(N_SLABS):
        p0 = wid * PROWS_PER_W + s * SLAB
        start_chunk(0, bufs_a, sem_a)
        start_chunk(1, bufs_b, sem_b)
        start_row(2 * p0, stg[0])
        for r in range(SLAB):
            p = p0 + r
            wait_row(stg[0])
            start_row(2 * p + 1, stg[1])
            pack_hi(stg[0], r)
            wait_row(stg[1])
            if r < SLAB - 1:
                start_row(2 * p + 2, stg[0])
            pack_lo(stg[1], r)

        def h_body(k, accs):
            base = k * 16
            hv = h_v[pl.ds(base, 16)]
            acc_a, acc_b = accs
            new_a, new_b = [], []
            for r in range(SLAB):
                w = xslab[pl.ds(r * N + base, 16)]
                av = plsc.bitcast(w & HIMASK, jnp.float32)
                bv = plsc.bitcast(w << 16, jnp.float32)
                new_a.append(acc_a[r] + av * hv)
                new_b.append(acc_b[r] + bv * hv)
            return tuple(new_a), tuple(new_b)

        zeros = tuple(jnp.zeros((16,), jnp.float32) for _ in range(SLAB))
        accs = lax.fori_loop(0, H_GROUPS, h_body, (zeros, zeros))

        def pair_body(c, accs):
            wait_chunk(bufs_a, sem_a)
            accs = edge_accum(bufs_a, accs)

            @pl.when(c < N_PAIRS - 1)
            def _():
                start_chunk(2 * c + 2, bufs_a, sem_a)

            wait_chunk(bufs_b, sem_b)
            accs = edge_accum(bufs_b, accs)

            @pl.when(c < N_PAIRS - 1)
            def _():
                start_chunk(2 * c + 3, bufs_b, sem_b)

            return accs

        acc_a, acc_b = lax.fori_loop(0, N_PAIRS, pair_body, accs)
        for r in range(SLAB):
            tile_sums[16 * s + 2 * r] = lax.reduce_sum_p.bind(
                acc_a[r], axes=(0,))
            tile_sums[16 * s + 2 * r + 1] = lax.reduce_sum_p.bind(
                acc_b[r], axes=(0,))

    for half in range(ROWS_PER_W // 16):
        vec = jnp.zeros((16,), jnp.float32)
        for k in range(16):
            vec = jnp.where(lane == k, tile_sums[half * 16 + k], vec)
        out_v[pl.ds(half * 16, 16)] = vec
    pltpu.sync_copy(out_v, out_hbm.at[pl.ds(wid * ROWS_PER_W, ROWS_PER_W)])


def kernel(x, h, J, edge_idx_i, edge_idx_j):
    return _rbm_sc(x.reshape(-1), h, J, edge_idx_i, edge_idx_j)
